# Initial kernel scaffold; baseline (speedup 1.0000x reference)
import jax
import jax.numpy as jnp
from jax.experimental import pallas as pl


def kernel(users, pos_items, neg_items, src, dst, entity_table, user_table):
    raise NotImplementedError("write your pallas kernel here")


# speedup vs baseline: 14.1952x; 14.1952x over previous
import functools

import jax
import jax.numpy as jnp
from jax import lax
from jax.experimental import pallas as pl
from jax.experimental.pallas import tpu as pltpu
from jax.experimental.pallas import tpu_sc as plsc

NUM_ENTITY = 100000
DIM = 32
N_EDGES = 1600000
BATCH = 4096

NC, NS = 2, 16
NW = NC * NS
HALF = NUM_ENTITY // NC
TRASH = 1024
ACC_ROWS = 51200

SUB = 128
NSUB = 4
CHUNK = SUB * NSUB
MACROS = 98
EDGE_PAD = NW * MACROS * CHUNK
TILE_EDGE_ROWS = MACROS * NSUB

DEG_CHUNK = 10000
DEG_CHUNKS = N_EDGES // (NW * DEG_CHUNK)

NORM_TILES = 25
NORM_ROWS = 4000
NORM_CHUNK = 800

SC_ROWS = NUM_ENTITY // NW
SC_CHUNK = 625

_mesh = plsc.VectorSubcoreMesh(core_axis_name="c", subcore_axis_name="s")
_SC_PARAMS = pltpu.CompilerParams(needs_layout_passes=False,
                                  use_tc_tiling_on_sc=False)


def _wid():
    return lax.axis_index("c") * NS + lax.axis_index("s")


def _deg_body(src_hbm, dst_hbm, deg_hbm, cnt, ibuf):
    wid = _wid()
    z = jnp.zeros((16,), jnp.float32)

    def zero_cnt(i, _):
        cnt[pl.ds(i * 16, 16)] = z
        return 0

    lax.fori_loop(0, NUM_ENTITY // 16, zero_cnt, 0)

    ones = jnp.ones((16,), jnp.float32)

    def count_chunks(edge_hbm):
        def chunk(k, _):
            pltpu.sync_copy(
                edge_hbm.at[pl.ds(wid * (N_EDGES // NW) + k * DEG_CHUNK,
                                  DEG_CHUNK)],
                ibuf)

            def inner(i, _):
                idx = ibuf[pl.ds(i * 16, 16)]
                plsc.addupdate_scatter(cnt, [idx], ones)
                return 0

            lax.fori_loop(0, DEG_CHUNK // 16, inner, 0)
            return 0

        lax.fori_loop(0, DEG_CHUNKS, chunk, 0)

    count_chunks(src_hbm)
    count_chunks(dst_hbm)
    pltpu.sync_copy(cnt, deg_hbm.at[wid])


def _deg_call(src, dst):
    return pl.kernel(
        _deg_body,
        out_type=jax.ShapeDtypeStruct((NW, NUM_ENTITY), jnp.float32),
        mesh=_mesh,
        compiler_params=_SC_PARAMS,
        scratch_types=[
            pltpu.VMEM((NUM_ENTITY,), jnp.float32),
            pltpu.VMEM((DEG_CHUNK,), jnp.int32),
        ],
    )(src, dst)


def _rsqrt16(d):
    i = plsc.bitcast(d, jnp.int32)
    i = jnp.int32(0x5F3759DF) - lax.shift_right_arithmetic(i, 1)
    y = plsc.bitcast(i, jnp.float32)
    half = d * 0.5
    for _ in range(4):
        y = y * (1.5 - half * y * y)
    return y


def _norm_body(deg_hbm, ent_hbm, nrep_hbm, w0_hbm, db, dt, nb, eb, nrb, wb):
    wid = _wid()

    @pl.when(wid < NORM_TILES)
    def _():
        row0 = wid * NORM_ROWS
        pltpu.sync_copy(deg_hbm.at[0, pl.ds(row0, NORM_ROWS)], db)
        for p in range(1, NW):
            pltpu.sync_copy(deg_hbm.at[p, pl.ds(row0, NORM_ROWS)], dt)

            def acc_part(i, _):
                db[pl.ds(i * 16, 16)] = (db[pl.ds(i * 16, 16)]
                                         + dt[pl.ds(i * 16, 16)])
                return 0

            lax.fori_loop(0, NORM_ROWS // 16, acc_part, 0)

        def newton(i, _):
            d = jnp.maximum(db[pl.ds(i * 16, 16)], 1.0)
            nb[pl.ds(i * 16, 16)] = _rsqrt16(d)
            return 0

        lax.fori_loop(0, NORM_ROWS // 16, newton, 0)

        for k in range(NORM_ROWS // NORM_CHUNK):
            r0 = row0 + k * NORM_CHUNK
            pltpu.sync_copy(ent_hbm.at[pl.ds(r0, NORM_CHUNK)], eb)

            def expand(r, _):
                bc = plsc.load_gather(
                    nb, [jnp.full((16,), k * NORM_CHUNK + r, jnp.int32)])
                for h in (0, 16):
                    nrb[r, pl.ds(h, 16)] = bc
                    wb[r, pl.ds(h, 16)] = eb[r, pl.ds(h, 16)] * bc
                return 0

            lax.fori_loop(0, NORM_CHUNK, expand, 0)
            pltpu.sync_copy(nrb, nrep_hbm.at[pl.ds(r0, NORM_CHUNK)])
            pltpu.sync_copy(wb, w0_hbm.at[pl.ds(r0, NORM_CHUNK)])


def _norm_call(deg, entity):
    return pl.kernel(
        _norm_body,
        out_type=(jax.ShapeDtypeStruct((NUM_ENTITY, DIM), jnp.float32),
                  jax.ShapeDtypeStruct((NUM_ENTITY, DIM), jnp.float32)),
        mesh=_mesh,
        compiler_params=_SC_PARAMS,
        scratch_types=[
            pltpu.VMEM((NORM_ROWS,), jnp.float32),
            pltpu.VMEM((NORM_ROWS,), jnp.float32),
            pltpu.VMEM((NORM_ROWS,), jnp.float32),
            pltpu.VMEM((NORM_CHUNK, DIM), jnp.float32),
            pltpu.VMEM((NORM_CHUNK, DIM), jnp.float32),
            pltpu.VMEM((NORM_CHUNK, DIM), jnp.float32),
        ],
    )(deg, entity)


def _spmm_body(w_hbm, src_hbm, dst_hbm, agg_hbm, sidx, didx, rows, acc, sem):
    c = lax.axis_index("c")
    s = lax.axis_index("s")
    wid = c * NS + s

    z = jnp.zeros((16,), jnp.float32)

    def zrow(r, _):
        rows[r, pl.ds(0, 16)] = z
        rows[r, pl.ds(16, 16)] = z
        return 0

    lax.fori_loop(0, CHUNK, zrow, 0)
    zbase = s * (ACC_ROWS // NS)
    for j in range(6):
        pltpu.sync_copy(rows.at[pl.ds(0, CHUNK)],
                        acc.at[pl.ds(zbase + j * CHUNK, CHUNK)])
    pltpu.sync_copy(rows.at[pl.ds(0, 128)], acc.at[pl.ds(zbase + 3072, 128)])
    plsc.subcore_barrier()

    node_base = c * HALF
    row0 = wid * TILE_EDGE_ROWS

    def macro(m, _):
        r = row0 + m * NSUB
        pltpu.sync_copy(src_hbm.at[pl.ds(r, NSUB)], sidx)
        pltpu.sync_copy(dst_hbm.at[pl.ds(r, NSUB)], didx)
        for j in range(NSUB):
            for i in range(SUB // 16):
                v = didx[j, pl.ds(i * 16, 16)]
                loc = v - node_base
                valid = (loc >= 0) & (loc < HALF)
                tr = HALF + (v & (TRASH - 1))
                didx[j, pl.ds(i * 16, 16)] = jnp.where(valid, loc, tr)
        descs = [
            pltpu.async_copy(w_hbm.at[sidx.at[j]],
                             rows.at[pl.ds(j * SUB, SUB)], sem)
            for j in range(NSUB)
        ]
        for d in descs:
            d.wait()
        for j in range(NSUB):
            pltpu.sync_copy(rows.at[pl.ds(j * SUB, SUB)],
                            acc.at[didx.at[j]], add=True)
        return 0

    lax.fori_loop(0, MACROS, macro, 0)
    plsc.subcore_barrier()

    pltpu.sync_copy(acc.at[pl.ds(s * (HALF // NS), HALF // NS)],
                    agg_hbm.at[pl.ds(c * HALF + s * (HALF // NS), HALF // NS)])


def _spmm_call(w, src2d, dst2d):
    return pl.kernel(
        _spmm_body,
        out_type=jax.ShapeDtypeStruct((NUM_ENTITY, DIM), jnp.float32),
        mesh=_mesh,
        compiler_params=_SC_PARAMS,
        scratch_types=[
            pltpu.VMEM((NSUB, SUB), jnp.int32),
            pltpu.VMEM((NSUB, SUB), jnp.int32),
            pltpu.VMEM((CHUNK, DIM), jnp.float32),
            pltpu.VMEM_SHARED((ACC_ROWS, DIM), jnp.float32),
            pltpu.SemaphoreType.DMA,
        ],
    )(w, src2d, dst2d)


def _scale_body(is_last, agg_hbm, nrep_hbm, s_hbm, *refs):
    if is_last:
        (sout_hbm, ab, nb, sb, wb) = refs
        wout_hbm = None
    else:
        (sout_hbm, wout_hbm, ab, nb, sb, wb) = refs
    wid = _wid()

    for k in range(SC_ROWS // SC_CHUNK):
        g0 = wid * SC_ROWS + k * SC_CHUNK
        pltpu.sync_copy(agg_hbm.at[pl.ds(g0, SC_CHUNK)], ab)
        pltpu.sync_copy(nrep_hbm.at[pl.ds(g0, SC_CHUNK)], nb)
        pltpu.sync_copy(s_hbm.at[pl.ds(g0, SC_CHUNK)], sb)

        def row(r, _):
            for h in (0, 16):
                n = nb[r, pl.ds(h, 16)]
                zv = ab[r, pl.ds(h, 16)] * n
                snew = sb[r, pl.ds(h, 16)] + zv
                if is_last:
                    sb[r, pl.ds(h, 16)] = snew * 0.25
                else:
                    sb[r, pl.ds(h, 16)] = snew
                    wb[r, pl.ds(h, 16)] = zv * n
            return 0

        lax.fori_loop(0, SC_CHUNK, row, 0)
        pltpu.sync_copy(sb, sout_hbm.at[pl.ds(g0, SC_CHUNK)])
        if not is_last:
            pltpu.sync_copy(wb, wout_hbm.at[pl.ds(g0, SC_CHUNK)])


def _scale_call(agg, nrep, s_in, is_last):
    sds = jax.ShapeDtypeStruct((NUM_ENTITY, DIM), jnp.float32)
    out_type = sds if is_last else (sds, sds)
    return pl.kernel(
        functools.partial(_scale_body, is_last),
        out_type=out_type,
        mesh=_mesh,
        compiler_params=_SC_PARAMS,
        scratch_types=[
            pltpu.VMEM((SC_CHUNK, DIM), jnp.float32),
            pltpu.VMEM((SC_CHUNK, DIM), jnp.float32),
            pltpu.VMEM((SC_CHUNK, DIM), jnp.float32),
            pltpu.VMEM((SC_CHUNK, DIM), jnp.float32),
        ],
    )(agg, nrep, s_in)


def _gather_body(gout_hbm, utab_hbm, pos_hbm, neg_hbm, usr_hbm,
                 pe_hbm, ne_hbm, ue_hbm, idxb, rowb, sem):
    wid = _wid()
    base = wid * (BATCH // NW)
    for tab, idx_hbm, out_hbm in ((gout_hbm, pos_hbm, pe_hbm),
                                  (gout_hbm, neg_hbm, ne_hbm),
                                  (utab_hbm, usr_hbm, ue_hbm)):
        pltpu.sync_copy(idx_hbm.at[pl.ds(base, BATCH // NW)], idxb)
        pltpu.async_copy(tab.at[idxb], rowb, sem).wait()
        pltpu.sync_copy(rowb, out_hbm.at[pl.ds(base, BATCH // NW)])


def _gather_call(gout, utab, pos, neg, usr):
    sds = jax.ShapeDtypeStruct((BATCH, DIM), jnp.float32)
    return pl.kernel(
        _gather_body,
        out_type=(sds, sds, sds),
        mesh=_mesh,
        compiler_params=_SC_PARAMS,
        scratch_types=[
            pltpu.VMEM((BATCH // NW,), jnp.int32),
            pltpu.VMEM((BATCH // NW, DIM), jnp.float32),
            pltpu.SemaphoreType.DMA,
        ],
    )(gout, utab, pos, neg, usr)


def _loss_body(u_ref, p_ref, n_ref, out_ref):
    u = u_ref[...]
    pos = jnp.sum(u * p_ref[...], axis=1)
    neg = jnp.sum(u * n_ref[...], axis=1)
    x = neg - pos
    out_ref[...] = (jnp.maximum(x, 0.0)
                    + jnp.log1p(jnp.exp(-jnp.abs(x))))[:, None]


def _loss_call(ue, pe, ne):
    return pl.pallas_call(
        _loss_body,
        out_shape=jax.ShapeDtypeStruct((BATCH, 1), jnp.float32),
    )(ue, pe, ne)


def kernel(users, pos_items, neg_items, src, dst, entity_table, user_table):
    users = users.astype(jnp.int32)
    pos_items = pos_items.astype(jnp.int32)
    neg_items = neg_items.astype(jnp.int32)
    src = src.astype(jnp.int32)
    dst = dst.astype(jnp.int32)

    deg = _deg_call(src, dst)
    nrep, w = _norm_call(deg, entity_table)

    pad = EDGE_PAD - N_EDGES
    src2d = jnp.pad(src, (0, pad)).reshape(EDGE_PAD // SUB, SUB)
    dst2d = jnp.pad(dst, (0, pad),
                    constant_values=NUM_ENTITY).reshape(EDGE_PAD // SUB, SUB)

    s_acc = entity_table
    for layer in range(3):
        agg = _spmm_call(w, src2d, dst2d)
        if layer < 2:
            s_acc, w = _scale_call(agg, nrep, s_acc, False)
        else:
            s_acc = _scale_call(agg, nrep, s_acc, True)

    pe, ne, ue = _gather_call(s_acc, user_table, pos_items, neg_items, users)
    loss = _loss_call(ue, pe, ne)
    return loss.reshape(BATCH)


# speedup vs baseline: 19.3102x; 1.3603x over previous
import functools

import jax
import jax.numpy as jnp
from jax import lax
from jax.experimental import pallas as pl
from jax.experimental.pallas import tpu as pltpu
from jax.experimental.pallas import tpu_sc as plsc

NUM_ENTITY = 100000
DIM = 32
N_EDGES = 1600000
BATCH = 4096

NC, NS = 2, 16
NW = NC * NS
HALF = NUM_ENTITY // NC
TRASH = 1024
ACC_ROWS = 51200

SUB = 128
BLK = 8
BLOCKS = 49
EDGE_PAD = NW * BLOCKS * BLK * SUB
TILE_EDGE_ROWS = BLOCKS * BLK
RING = 4

DEG_CHUNK = 10000
DEG_CHUNKS = N_EDGES // (NW * DEG_CHUNK)

NORM_TILES = 25
NORM_ROWS = 4000
NORM_CHUNK = 800

SC_ROWS = NUM_ENTITY // NW
SC_CHUNK = 625

_mesh = plsc.VectorSubcoreMesh(core_axis_name="c", subcore_axis_name="s")
_SC_PARAMS = pltpu.CompilerParams(needs_layout_passes=False,
                                  use_tc_tiling_on_sc=False)


def _wid():
    return lax.axis_index("c") * NS + lax.axis_index("s")


def _deg_body(src_hbm, dst_hbm, deg_hbm, cnt, ibuf):
    wid = _wid()
    z = jnp.zeros((16,), jnp.float32)

    def zero_cnt(i, _):
        cnt[pl.ds(i * 16, 16)] = z
        return 0

    lax.fori_loop(0, NUM_ENTITY // 16, zero_cnt, 0)

    ones = jnp.ones((16,), jnp.float32)

    def count_chunks(edge_hbm):
        def chunk(k, _):
            pltpu.sync_copy(
                edge_hbm.at[pl.ds(wid * (N_EDGES // NW) + k * DEG_CHUNK,
                                  DEG_CHUNK)],
                ibuf)

            def inner(i, _):
                idx = ibuf[pl.ds(i * 16, 16)]
                plsc.addupdate_scatter(cnt, [idx], ones)
                return 0

            lax.fori_loop(0, DEG_CHUNK // 16, inner, 0)
            return 0

        lax.fori_loop(0, DEG_CHUNKS, chunk, 0)

    count_chunks(src_hbm)
    count_chunks(dst_hbm)
    pltpu.sync_copy(cnt, deg_hbm.at[wid])


def _deg_call(src, dst):
    return pl.kernel(
        _deg_body,
        out_type=jax.ShapeDtypeStruct((NW, NUM_ENTITY), jnp.float32),
        mesh=_mesh,
        compiler_params=_SC_PARAMS,
        scratch_types=[
            pltpu.VMEM((NUM_ENTITY,), jnp.float32),
            pltpu.VMEM((DEG_CHUNK,), jnp.int32),
        ],
    )(src, dst)


def _rsqrt16(d):
    i = plsc.bitcast(d, jnp.int32)
    i = jnp.int32(0x5F3759DF) - lax.shift_right_arithmetic(i, 1)
    y = plsc.bitcast(i, jnp.float32)
    half = d * 0.5
    for _ in range(4):
        y = y * (1.5 - half * y * y)
    return y


def _norm_body(deg_hbm, ent_hbm, nrep_hbm, w0_hbm, db, dt, nb, eb, nrb, wb):
    wid = _wid()

    @pl.when(wid < NORM_TILES)
    def _():
        row0 = wid * NORM_ROWS
        pltpu.sync_copy(deg_hbm.at[0, pl.ds(row0, NORM_ROWS)], db)
        for p in range(1, NW):
            pltpu.sync_copy(deg_hbm.at[p, pl.ds(row0, NORM_ROWS)], dt)

            def acc_part(i, _):
                db[pl.ds(i * 16, 16)] = (db[pl.ds(i * 16, 16)]
                                         + dt[pl.ds(i * 16, 16)])
                return 0

            lax.fori_loop(0, NORM_ROWS // 16, acc_part, 0)

        def newton(i, _):
            d = jnp.maximum(db[pl.ds(i * 16, 16)], 1.0)
            nb[pl.ds(i * 16, 16)] = _rsqrt16(d)
            return 0

        lax.fori_loop(0, NORM_ROWS // 16, newton, 0)

        for k in range(NORM_ROWS // NORM_CHUNK):
            r0 = row0 + k * NORM_CHUNK
            pltpu.sync_copy(ent_hbm.at[pl.ds(r0, NORM_CHUNK)], eb)

            def expand(r, _):
                bc = plsc.load_gather(
                    nb, [jnp.full((16,), k * NORM_CHUNK + r, jnp.int32)])
                for h in (0, 16):
                    nrb[r, pl.ds(h, 16)] = bc
                    wb[r, pl.ds(h, 16)] = eb[r, pl.ds(h, 16)] * bc
                return 0

            lax.fori_loop(0, NORM_CHUNK, expand, 0)
            pltpu.sync_copy(nrb, nrep_hbm.at[pl.ds(r0, NORM_CHUNK)])
            pltpu.sync_copy(wb, w0_hbm.at[pl.ds(r0, NORM_CHUNK)])


def _norm_call(deg, entity):
    return pl.kernel(
        _norm_body,
        out_type=(jax.ShapeDtypeStruct((NUM_ENTITY, DIM), jnp.float32),
                  jax.ShapeDtypeStruct((NUM_ENTITY, DIM), jnp.float32)),
        mesh=_mesh,
        compiler_params=_SC_PARAMS,
        scratch_types=[
            pltpu.VMEM((NORM_ROWS,), jnp.float32),
            pltpu.VMEM((NORM_ROWS,), jnp.float32),
            pltpu.VMEM((NORM_ROWS,), jnp.float32),
            pltpu.VMEM((NORM_CHUNK, DIM), jnp.float32),
            pltpu.VMEM((NORM_CHUNK, DIM), jnp.float32),
            pltpu.VMEM((NORM_CHUNK, DIM), jnp.float32),
        ],
    )(deg, entity)


def _spmm_body(w_hbm, src_hbm, dst_hbm, agg_hbm,
               sidx_a, didx_a, sidx_b, didx_b, rows, acc, gsem, ssem, isem):
    c = lax.axis_index("c")
    s = lax.axis_index("s")
    wid = c * NS + s

    z = jnp.zeros((16,), jnp.float32)

    def zrow(r, _):
        rows[r, pl.ds(0, 16)] = z
        rows[r, pl.ds(16, 16)] = z
        return 0

    lax.fori_loop(0, RING * SUB, zrow, 0)
    zbase = s * (ACC_ROWS // NS)
    for j in range(6):
        pltpu.sync_copy(rows.at[pl.ds(0, RING * SUB)],
                        acc.at[pl.ds(zbase + j * RING * SUB, RING * SUB)])
    pltpu.sync_copy(rows.at[pl.ds(0, 128)], acc.at[pl.ds(zbase + 3072, 128)])
    plsc.subcore_barrier()

    node_base = c * HALF
    trash_base = HALF + s * (TRASH // NS)
    row0 = wid * TILE_EDGE_ROWS

    def do_block(m, sidx, didx, sidx_next, didx_next):
        @pl.when(m + 1 < BLOCKS)
        def _():
            r_next = row0 + (m + 1) * BLK
            pltpu.async_copy(src_hbm.at[pl.ds(r_next, BLK)], sidx_next, isem)
            pltpu.async_copy(dst_hbm.at[pl.ds(r_next, BLK)], didx_next, isem)

        for j in range(BLK):
            for i in range(SUB // 16):
                v = didx[j, pl.ds(i * 16, 16)]
                loc = v - node_base
                valid = (loc >= 0) & (loc < HALF)
                tr = trash_base + (v & (TRASH // NS - 1))
                didx[j, pl.ds(i * 16, 16)] = jnp.where(valid, loc, tr)

        g = {}
        sv = {}
        for j in range(RING):
            g[j] = pltpu.async_copy(w_hbm.at[sidx.at[j]],
                                    rows.at[pl.ds(j * SUB, SUB)], gsem)
        for j in range(BLK):
            p = j % RING
            g[j].wait()
            sv[j] = pltpu.async_copy(rows.at[pl.ds(p * SUB, SUB)],
                                     acc.at[didx.at[j]], ssem, add=True)
            jj = j - 2
            if 0 <= jj and jj + RING < BLK:
                sv[jj].wait()
                g[jj + RING] = pltpu.async_copy(
                    w_hbm.at[sidx.at[jj + RING]],
                    rows.at[pl.ds((jj % RING) * SUB, SUB)], gsem)
        for j in range(BLK - RING, BLK):
            sv[j].wait()

        @pl.when(m + 1 < BLOCKS)
        def _():
            pltpu.make_async_copy(src_hbm.at[pl.ds(0, BLK)], sidx_next,
                                  isem).wait()
            pltpu.make_async_copy(dst_hbm.at[pl.ds(0, BLK)], didx_next,
                                  isem).wait()

    pltpu.sync_copy(src_hbm.at[pl.ds(row0, BLK)], sidx_a)
    pltpu.sync_copy(dst_hbm.at[pl.ds(row0, BLK)], didx_a)

    def pair(k, _):
        m = k * 2
        do_block(m, sidx_a, didx_a, sidx_b, didx_b)
        do_block(m + 1, sidx_b, didx_b, sidx_a, didx_a)
        return 0

    lax.fori_loop(0, BLOCKS // 2, pair, 0)
    do_block(BLOCKS - 1, sidx_a, didx_a, sidx_b, didx_b)
    plsc.subcore_barrier()

    pltpu.sync_copy(acc.at[pl.ds(s * (HALF // NS), HALF // NS)],
                    agg_hbm.at[pl.ds(c * HALF + s * (HALF // NS), HALF // NS)])


def _spmm_call(w, src2d, dst2d):
    return pl.kernel(
        _spmm_body,
        out_type=jax.ShapeDtypeStruct((NUM_ENTITY, DIM), jnp.float32),
        mesh=_mesh,
        compiler_params=_SC_PARAMS,
        scratch_types=[
            pltpu.VMEM((BLK, SUB), jnp.int32),
            pltpu.VMEM((BLK, SUB), jnp.int32),
            pltpu.VMEM((BLK, SUB), jnp.int32),
            pltpu.VMEM((BLK, SUB), jnp.int32),
            pltpu.VMEM((RING * SUB, DIM), jnp.float32),
            pltpu.VMEM_SHARED((ACC_ROWS, DIM), jnp.float32),
            pltpu.SemaphoreType.DMA,
            pltpu.SemaphoreType.DMA,
            pltpu.SemaphoreType.DMA,
        ],
    )(w, src2d, dst2d)


def _scale_body(is_last, agg_hbm, nrep_hbm, s_hbm, *refs):
    if is_last:
        (sout_hbm, ab, nb, sb, wb) = refs
        wout_hbm = None
    else:
        (sout_hbm, wout_hbm, ab, nb, sb, wb) = refs
    wid = _wid()

    for k in range(SC_ROWS // SC_CHUNK):
        g0 = wid * SC_ROWS + k * SC_CHUNK
        pltpu.sync_copy(agg_hbm.at[pl.ds(g0, SC_CHUNK)], ab)
        pltpu.sync_copy(nrep_hbm.at[pl.ds(g0, SC_CHUNK)], nb)
        pltpu.sync_copy(s_hbm.at[pl.ds(g0, SC_CHUNK)], sb)

        def row(r, _):
            for h in (0, 16):
                n = nb[r, pl.ds(h, 16)]
                zv = ab[r, pl.ds(h, 16)] * n
                snew = sb[r, pl.ds(h, 16)] + zv
                if is_last:
                    sb[r, pl.ds(h, 16)] = snew * 0.25
                else:
                    sb[r, pl.ds(h, 16)] = snew
                    wb[r, pl.ds(h, 16)] = zv * n
            return 0

        lax.fori_loop(0, SC_CHUNK, row, 0)
        pltpu.sync_copy(sb, sout_hbm.at[pl.ds(g0, SC_CHUNK)])
        if not is_last:
            pltpu.sync_copy(wb, wout_hbm.at[pl.ds(g0, SC_CHUNK)])


def _scale_call(agg, nrep, s_in, is_last):
    sds = jax.ShapeDtypeStruct((NUM_ENTITY, DIM), jnp.float32)
    out_type = sds if is_last else (sds, sds)
    return pl.kernel(
        functools.partial(_scale_body, is_last),
        out_type=out_type,
        mesh=_mesh,
        compiler_params=_SC_PARAMS,
        scratch_types=[
            pltpu.VMEM((SC_CHUNK, DIM), jnp.float32),
            pltpu.VMEM((SC_CHUNK, DIM), jnp.float32),
            pltpu.VMEM((SC_CHUNK, DIM), jnp.float32),
            pltpu.VMEM((SC_CHUNK, DIM), jnp.float32),
        ],
    )(agg, nrep, s_in)


def _gather_body(gout_hbm, utab_hbm, pos_hbm, neg_hbm, usr_hbm,
                 pe_hbm, ne_hbm, ue_hbm, idxb, rowb, sem):
    wid = _wid()
    base = wid * (BATCH // NW)
    for tab, idx_hbm, out_hbm in ((gout_hbm, pos_hbm, pe_hbm),
                                  (gout_hbm, neg_hbm, ne_hbm),
                                  (utab_hbm, usr_hbm, ue_hbm)):
        pltpu.sync_copy(idx_hbm.at[pl.ds(base, BATCH // NW)], idxb)
        pltpu.async_copy(tab.at[idxb], rowb, sem).wait()
        pltpu.sync_copy(rowb, out_hbm.at[pl.ds(base, BATCH // NW)])


def _gather_call(gout, utab, pos, neg, usr):
    sds = jax.ShapeDtypeStruct((BATCH, DIM), jnp.float32)
    return pl.kernel(
        _gather_body,
        out_type=(sds, sds, sds),
        mesh=_mesh,
        compiler_params=_SC_PARAMS,
        scratch_types=[
            pltpu.VMEM((BATCH // NW,), jnp.int32),
            pltpu.VMEM((BATCH // NW, DIM), jnp.float32),
            pltpu.SemaphoreType.DMA,
        ],
    )(gout, utab, pos, neg, usr)


def _loss_body(u_ref, p_ref, n_ref, out_ref):
    u = u_ref[...]
    pos = jnp.sum(u * p_ref[...], axis=1)
    neg = jnp.sum(u * n_ref[...], axis=1)
    x = neg - pos
    out_ref[...] = (jnp.maximum(x, 0.0)
                    + jnp.log1p(jnp.exp(-jnp.abs(x))))[:, None]


def _loss_call(ue, pe, ne):
    return pl.pallas_call(
        _loss_body,
        out_shape=jax.ShapeDtypeStruct((BATCH, 1), jnp.float32),
    )(ue, pe, ne)


def kernel(users, pos_items, neg_items, src, dst, entity_table, user_table):
    users = users.astype(jnp.int32)
    pos_items = pos_items.astype(jnp.int32)
    neg_items = neg_items.astype(jnp.int32)
    src = src.astype(jnp.int32)
    dst = dst.astype(jnp.int32)

    deg = _deg_call(src, dst)
    nrep, w = _norm_call(deg, entity_table)

    pad = EDGE_PAD - N_EDGES
    src2d = jnp.pad(src, (0, pad)).reshape(EDGE_PAD // SUB, SUB)
    dst2d = jnp.pad(dst, (0, pad),
                    constant_values=NUM_ENTITY).reshape(EDGE_PAD // SUB, SUB)

    s_acc = entity_table
    for layer in range(3):
        agg = _spmm_call(w, src2d, dst2d)
        if layer < 2:
            s_acc, w = _scale_call(agg, nrep, s_acc, False)
        else:
            s_acc = _scale_call(agg, nrep, s_acc, True)

    pe, ne, ue = _gather_call(s_acc, user_table, pos_items, neg_items, users)
    loss = _loss_call(ue, pe, ne)
    return loss.reshape(BATCH)
